# baseline (speedup 1.0000x reference)
import functools

import jax
import jax.numpy as jnp
from jax import lax
from jax.experimental import pallas as pl
from jax.experimental.pallas import tpu as pltpu
from jax.experimental.pallas import tpu_sc as plsc

MAXLEN = 200
EMBED = 64
LANES = 16
VPR = EMBED // LANES

_info = plsc.get_sparse_core_info()
NC, NS = _info.num_cores, _info.num_subcores
NW = NC * NS

CHUNK_SEQS = 4
CHUNK_ROWS = CHUNK_SEQS * MAXLEN


def _body(x_hbm, tok_hbm, pos_hbm, out_hbm, pos_v, idx_v, rows_v, sem):
    wid = lax.axis_index("s") * NC + lax.axis_index("c")
    n_flat = x_hbm.shape[0]
    per_w = n_flat // NW
    n_chunks = per_w // CHUNK_ROWS
    base = wid * per_w

    pltpu.sync_copy(pos_hbm, pos_v)

    def chunk_body(g, carry):
        row0 = base + g * CHUNK_ROWS
        pltpu.sync_copy(x_hbm.at[pl.ds(row0, CHUNK_ROWS)], idx_v)
        pltpu.async_copy(tok_hbm.at[idx_v], rows_v, sem).wait()

        def add_body(p, c2):
            for j in range(VPR):
                pv = pos_v[p, pl.ds(j * LANES, LANES)]
                for s in range(CHUNK_SEQS):
                    r = s * MAXLEN + p
                    rows_v[r, pl.ds(j * LANES, LANES)] = (
                        rows_v[r, pl.ds(j * LANES, LANES)] + pv
                    )
            return c2

        lax.fori_loop(0, MAXLEN, add_body, 0, unroll=False)
        pltpu.sync_copy(rows_v, out_hbm.at[pl.ds(row0, CHUNK_ROWS)])
        return carry

    lax.fori_loop(0, n_chunks, chunk_body, 0, unroll=False)


@jax.jit
def kernel(x, token_table, pos_table):
    batch, seq_len = x.shape
    n_flat = batch * seq_len
    x_flat = x.reshape(n_flat).astype(jnp.int32)

    mesh = plsc.VectorSubcoreMesh(core_axis_name="c", subcore_axis_name="s")
    run = pl.kernel(
        _body,
        out_type=jax.ShapeDtypeStruct((n_flat, EMBED), jnp.float32),
        mesh=mesh,
        scratch_types=[
            pltpu.VMEM((MAXLEN, EMBED), jnp.float32),
            pltpu.VMEM((CHUNK_ROWS,), jnp.int32),
            pltpu.VMEM((CHUNK_ROWS, EMBED), jnp.float32),
            pltpu.SemaphoreType.DMA,
        ],
        compiler_params=pltpu.CompilerParams(use_tc_tiling_on_sc=False),
    )
    out = run(x_flat, token_table, pos_table)
    return out.reshape(batch, seq_len, EMBED)
